# baseline (speedup 1.0000x reference)
import functools

import jax
import jax.numpy as jnp
from jax import lax
from jax.experimental import pallas as pl
from jax.experimental.pallas import tpu as pltpu
from jax.experimental.pallas import tpu_sc as plsc

N = 10000
E = 320000
D_IN = 128
D_H = 128
D_OUT = 64

NC = 2
NS = 16
NW = NC * NS
BB = 125
NB = E // (NW * BB)
HB = 40
RPS = N // NS
DPAD = 10240
DCH = DPAD // NS

_mesh = plsc.VectorSubcoreMesh(
    core_axis_name="c", subcore_axis_name="s", num_cores=NC, num_subcores=NS
)


def _zero_vmem_2d(ref, rows, cols):
  zv = jnp.zeros((16,), jnp.float32)

  def body(i, carry):
    r = i // (cols // 16)
    c = i % (cols // 16)
    ref[r, pl.ds(c * 16, 16)] = zv
    return carry

  lax.fori_loop(0, rows * (cols // 16), body, 0)


def _deg_body(src_hbm, dst_hbm, out_hbm, sidx_v, didx_v, ones_v, zb_v,
              dsrc_sh, ddst_sh):
  c = lax.axis_index("c")
  s = lax.axis_index("s")
  w = c * NS + s

  def init_body(i, carry):
    zb_v[pl.ds(i * 16, 16)] = jnp.zeros((16,), jnp.float32)
    return carry

  lax.fori_loop(0, DCH // 16, init_body, 0)

  def ones_body(i, carry):
    ones_v[pl.ds(i * 16, 16)] = jnp.ones((16,), jnp.float32)
    return carry

  lax.fori_loop(0, BB // 16, ones_body, 0)
  ones_v[pl.ds(BB - 16, 16)] = jnp.ones((16,), jnp.float32)

  off = pl.multiple_of(s * DCH, 8)
  pltpu.sync_copy(zb_v, dsrc_sh.at[pl.ds(off, DCH)])
  pltpu.sync_copy(zb_v, ddst_sh.at[pl.ds(off, DCH)])
  pltpu.sync_copy(src_hbm.at[pl.ds(w * NB, NB)], sidx_v)
  pltpu.sync_copy(dst_hbm.at[pl.ds(w * NB, NB)], didx_v)
  plsc.subcore_barrier()

  def body(i, carry):
    pltpu.sync_copy(ones_v, dsrc_sh.at[sidx_v.at[i]], add=True)
    pltpu.sync_copy(ones_v, ddst_sh.at[didx_v.at[i]], add=True)
    return carry

  lax.fori_loop(0, NB, body, 0)
  plsc.subcore_barrier()
  pltpu.sync_copy(dsrc_sh.at[pl.ds(off, DCH)], out_hbm.at[c, 0, pl.ds(off, DCH)])
  pltpu.sync_copy(ddst_sh.at[pl.ds(off, DCH)], out_hbm.at[c, 1, pl.ds(off, DCH)])


_deg_call = pl.kernel(
    _deg_body,
    out_type=jax.ShapeDtypeStruct((NC, 2, DPAD), jnp.float32),
    mesh=_mesh,
    scratch_types=[
        pltpu.VMEM((NB, BB), jnp.int32),
        pltpu.VMEM((NB, BB), jnp.int32),
        pltpu.VMEM((BB,), jnp.float32),
        pltpu.VMEM((DCH,), jnp.float32),
        pltpu.VMEM_SHARED((DPAD,), jnp.float32),
        pltpu.VMEM_SHARED((DPAD,), jnp.float32),
    ],
)


def _agg_body(d, h_hbm, src_hbm, dst_hbm, out_hbm, sidx_v, didx_v, rows_v,
              acc_sh, gsem0, gsem1):
  c = lax.axis_index("c")
  s = lax.axis_index("s")
  w = c * NS + s
  gsems = (gsem0, gsem1)

  _zero_vmem_2d(rows_v.at[0], BB, d)
  for j in range(RPS // BB):
    pltpu.sync_copy(rows_v.at[0], acc_sh.at[pl.ds(s * RPS + j * BB, BB)])
  plsc.subcore_barrier()

  for p in range(NB // HB):
    pltpu.sync_copy(src_hbm.at[pl.ds(w * NB + p * HB, HB)], sidx_v)
    pltpu.sync_copy(dst_hbm.at[pl.ds(w * NB + p * HB, HB)], didx_v)
    pltpu.async_copy(h_hbm.at[sidx_v.at[0]], rows_v.at[0], gsems[0])
    pltpu.async_copy(h_hbm.at[sidx_v.at[1]], rows_v.at[1], gsems[1])

    def body(jj, carry):
      j = jj * 2
      for b in range(2):
        i = j + b
        pltpu.make_async_copy(h_hbm.at[sidx_v.at[i]], rows_v.at[b],
                              gsems[b]).wait()
        pltpu.sync_copy(rows_v.at[b], acc_sh.at[didx_v.at[i]], add=True)

        @pl.when(i + 2 < HB)
        def _():
          pltpu.async_copy(h_hbm.at[sidx_v.at[i + 2]], rows_v.at[b], gsems[b])

      return carry

    lax.fori_loop(0, HB // 2, body, 0)
  plsc.subcore_barrier()
  pltpu.sync_copy(acc_sh.at[pl.ds(s * RPS, RPS)],
                  out_hbm.at[c, pl.ds(s * RPS, RPS)])


def _make_agg(d):
  return pl.kernel(
      functools.partial(_agg_body, d),
      out_type=jax.ShapeDtypeStruct((NC, N, d), jnp.float32),
      mesh=_mesh,
      scratch_types=[
          pltpu.VMEM((HB, BB), jnp.int32),
          pltpu.VMEM((HB, BB), jnp.int32),
          pltpu.VMEM((2, BB, d), jnp.float32),
          pltpu.VMEM_SHARED((N, d), jnp.float32),
          pltpu.SemaphoreType.DMA,
          pltpu.SemaphoreType.DMA,
      ],
      compiler_params=pltpu.CompilerParams(use_tc_tiling_on_sc=False),
  )


_agg_h = _make_agg(D_H)
_agg_o = _make_agg(D_OUT)


_BM = 1280
_GRID = (N + _BM - 1) // _BM


def _tc1_body(x_ref, w_ref, dp_ref, o_ref):
  d = dp_ref[0, :] + dp_ref[1, :]
  ns = lax.rsqrt(jnp.maximum(d, 1.0))
  y = jnp.dot(x_ref[...], w_ref[...], preferred_element_type=jnp.float32)
  o_ref[...] = y * ns[:, None]


def _tc1(x, w1, deg_src_p):
  return pl.pallas_call(
      _tc1_body,
      grid=(_GRID,),
      in_specs=[
          pl.BlockSpec((_BM, D_IN), lambda i: (i, 0)),
          pl.BlockSpec((D_IN, D_H), lambda i: (0, 0)),
          pl.BlockSpec((NC, _BM), lambda i: (0, i)),
      ],
      out_specs=pl.BlockSpec((_BM, D_H), lambda i: (i, 0)),
      out_shape=jax.ShapeDtypeStruct((N, D_H), jnp.float32),
  )(x, w1, deg_src_p)


def _tc2_body(p_ref, dd_ref, ds_ref, b1_ref, w2_ref, o_ref):
  t = p_ref[0] + p_ref[1]
  dd = dd_ref[0, :] + dd_ref[1, :]
  nd = lax.rsqrt(jnp.maximum(dd, 1.0))
  t = jnp.maximum(t * nd[:, None] + b1_ref[...], 0.0)
  dsum = ds_ref[0, :] + ds_ref[1, :]
  ns = lax.rsqrt(jnp.maximum(dsum, 1.0))
  y = jnp.dot(t, w2_ref[...], preferred_element_type=jnp.float32)
  o_ref[...] = y * ns[:, None]


def _tc2(p, deg_dst_p, deg_src_p, b1, w2):
  return pl.pallas_call(
      _tc2_body,
      grid=(_GRID,),
      in_specs=[
          pl.BlockSpec((NC, _BM, D_H), lambda i: (0, i, 0)),
          pl.BlockSpec((NC, _BM), lambda i: (0, i)),
          pl.BlockSpec((NC, _BM), lambda i: (0, i)),
          pl.BlockSpec((1, D_H), lambda i: (0, 0)),
          pl.BlockSpec((D_H, D_OUT), lambda i: (0, 0)),
      ],
      out_specs=pl.BlockSpec((_BM, D_OUT), lambda i: (i, 0)),
      out_shape=jax.ShapeDtypeStruct((N, D_OUT), jnp.float32),
  )(p, deg_dst_p, deg_src_p, b1, w2)


def _tc3_body(q_ref, dd_ref, b2_ref, o_ref):
  t = q_ref[0] + q_ref[1]
  dd = dd_ref[0, :] + dd_ref[1, :]
  nd = lax.rsqrt(jnp.maximum(dd, 1.0))
  o_ref[...] = t * nd[:, None] + b2_ref[...]


def _tc3(q, deg_dst_p, b2):
  return pl.pallas_call(
      _tc3_body,
      grid=(_GRID,),
      in_specs=[
          pl.BlockSpec((NC, _BM, D_OUT), lambda i: (0, i, 0)),
          pl.BlockSpec((NC, _BM), lambda i: (0, i)),
          pl.BlockSpec((1, D_OUT), lambda i: (0, 0)),
      ],
      out_specs=pl.BlockSpec((_BM, D_OUT), lambda i: (i, 0)),
      out_shape=jax.ShapeDtypeStruct((N, D_OUT), jnp.float32),
  )(q, deg_dst_p, b2)


def kernel(in_feat, edge_index, W1, b1, W2, b2):
  src2d = edge_index[0].reshape(NW * NB, BB)
  dst2d = edge_index[1].reshape(NW * NB, BB)
  degp = _deg_call(src2d, dst2d)
  deg_src_p = degp[:, 0, :]
  deg_dst_p = degp[:, 1, :]
  h1 = _tc1(in_feat, W1, deg_src_p)
  p1 = _agg_h(h1, src2d, dst2d)
  h2 = _tc2(p1, deg_dst_p, deg_src_p, b1.reshape(1, -1), W2)
  p2 = _agg_o(h2, src2d, dst2d)
  return _tc3(p2, deg_dst_p, b2.reshape(1, -1))


# speedup vs baseline: 1.0051x; 1.0051x over previous
import functools

import jax
import jax.numpy as jnp
from jax import lax
from jax.experimental import pallas as pl
from jax.experimental.pallas import tpu as pltpu
from jax.experimental.pallas import tpu_sc as plsc

N = 10000
E = 320000
D_IN = 128
D_H = 128
D_OUT = 64

NC = 2
NS = 16
NW = NC * NS
BB = 128
NB = 80
E_PAD = NW * NB * BB
HB = 40
NP = 10240
RPS = NP // NS

_mesh = plsc.VectorSubcoreMesh(
    core_axis_name="c", subcore_axis_name="s", num_cores=NC, num_subcores=NS
)


def _zero_vmem_2d(ref, rows, cols):
  zv = jnp.zeros((16,), jnp.float32)

  def body(i, carry):
    r = i // (cols // 16)
    c = i % (cols // 16)
    ref[r, pl.ds(c * 16, 16)] = zv
    return carry

  lax.fori_loop(0, rows * (cols // 16), body, 0)


def _deg_body(src_hbm, dst_hbm, out_hbm, sidx_v, didx_v, ones_v, zb_v,
              dsrc_sh, ddst_sh):
  c = lax.axis_index("c")
  s = lax.axis_index("s")
  w = c * NS + s

  def init_body(i, carry):
    zb_v[pl.ds(i * 16, 16)] = jnp.zeros((16,), jnp.float32)
    return carry

  lax.fori_loop(0, RPS // 16, init_body, 0)

  def ones_body(i, carry):
    ones_v[pl.ds(i * 16, 16)] = jnp.ones((16,), jnp.float32)
    return carry

  lax.fori_loop(0, BB // 16, ones_body, 0)

  off = pl.multiple_of(s * RPS, 8)
  pltpu.sync_copy(zb_v, dsrc_sh.at[pl.ds(off, RPS)])
  pltpu.sync_copy(zb_v, ddst_sh.at[pl.ds(off, RPS)])
  pltpu.sync_copy(src_hbm.at[pl.ds(w * NB, NB)], sidx_v)
  pltpu.sync_copy(dst_hbm.at[pl.ds(w * NB, NB)], didx_v)
  plsc.subcore_barrier()

  def body(i, carry):
    pltpu.sync_copy(ones_v, dsrc_sh.at[sidx_v.at[i]], add=True)
    pltpu.sync_copy(ones_v, ddst_sh.at[didx_v.at[i]], add=True)
    return carry

  lax.fori_loop(0, NB, body, 0)
  plsc.subcore_barrier()
  pltpu.sync_copy(dsrc_sh.at[pl.ds(off, RPS)], out_hbm.at[c, 0, pl.ds(off, RPS)])
  pltpu.sync_copy(ddst_sh.at[pl.ds(off, RPS)], out_hbm.at[c, 1, pl.ds(off, RPS)])


_deg_call = pl.kernel(
    _deg_body,
    out_type=jax.ShapeDtypeStruct((NC, 2, NP), jnp.float32),
    mesh=_mesh,
    scratch_types=[
        pltpu.VMEM((NB, BB), jnp.int32),
        pltpu.VMEM((NB, BB), jnp.int32),
        pltpu.VMEM((BB,), jnp.float32),
        pltpu.VMEM((RPS,), jnp.float32),
        pltpu.VMEM_SHARED((NP,), jnp.float32),
        pltpu.VMEM_SHARED((NP,), jnp.float32),
    ],
)


def _agg_body(d, h_hbm, src_hbm, dst_hbm, out_hbm, sidx_v, didx_v, rows_v,
              acc_sh, gsem0, gsem1):
  c = lax.axis_index("c")
  s = lax.axis_index("s")
  w = c * NS + s
  gsems = (gsem0, gsem1)

  _zero_vmem_2d(rows_v.at[0], BB, d)
  for j in range(RPS // BB):
    pltpu.sync_copy(rows_v.at[0], acc_sh.at[pl.ds(s * RPS + j * BB, BB)])
  plsc.subcore_barrier()

  for p in range(NB // HB):
    pltpu.sync_copy(src_hbm.at[pl.ds(w * NB + p * HB, HB)], sidx_v)
    pltpu.sync_copy(dst_hbm.at[pl.ds(w * NB + p * HB, HB)], didx_v)
    pltpu.async_copy(h_hbm.at[sidx_v.at[0]], rows_v.at[0], gsems[0])
    pltpu.async_copy(h_hbm.at[sidx_v.at[1]], rows_v.at[1], gsems[1])

    def body(jj, carry):
      j = jj * 2
      for b in range(2):
        i = j + b
        pltpu.make_async_copy(h_hbm.at[sidx_v.at[i]], rows_v.at[b],
                              gsems[b]).wait()
        pltpu.sync_copy(rows_v.at[b], acc_sh.at[didx_v.at[i]], add=True)

        @pl.when(i + 2 < HB)
        def _():
          pltpu.async_copy(h_hbm.at[sidx_v.at[i + 2]], rows_v.at[b], gsems[b])

      return carry

    lax.fori_loop(0, HB // 2, body, 0)
  plsc.subcore_barrier()
  pltpu.sync_copy(acc_sh.at[pl.ds(s * RPS, RPS)],
                  out_hbm.at[c, pl.ds(s * RPS, RPS)])


def _make_agg(d):
  return pl.kernel(
      functools.partial(_agg_body, d),
      out_type=jax.ShapeDtypeStruct((NC, NP, d), jnp.float32),
      mesh=_mesh,
      scratch_types=[
          pltpu.VMEM((HB, BB), jnp.int32),
          pltpu.VMEM((HB, BB), jnp.int32),
          pltpu.VMEM((2, BB, d), jnp.float32),
          pltpu.VMEM_SHARED((NP, d), jnp.float32),
          pltpu.SemaphoreType.DMA,
          pltpu.SemaphoreType.DMA,
      ],
      compiler_params=pltpu.CompilerParams(use_tc_tiling_on_sc=False),
  )


_agg_h = _make_agg(D_H)
_agg_o = _make_agg(D_OUT)


_BM = 1280
_GRID = NP // _BM


def _tc1_body(x_ref, w_ref, dp_ref, o_ref):
  d = dp_ref[0, :] + dp_ref[1, :]
  ns = lax.rsqrt(jnp.maximum(d, 1.0))
  y = jnp.dot(x_ref[...], w_ref[...], preferred_element_type=jnp.float32)
  o_ref[...] = y * ns[:, None]


def _tc1(x, w1, deg_src_p):
  return pl.pallas_call(
      _tc1_body,
      grid=(_GRID,),
      in_specs=[
          pl.BlockSpec((_BM, D_IN), lambda i: (i, 0)),
          pl.BlockSpec((D_IN, D_H), lambda i: (0, 0)),
          pl.BlockSpec((NC, _BM), lambda i: (0, i)),
      ],
      out_specs=pl.BlockSpec((_BM, D_H), lambda i: (i, 0)),
      out_shape=jax.ShapeDtypeStruct((NP, D_H), jnp.float32),
  )(x, w1, deg_src_p)


def _tc2_body(p_ref, dd_ref, ds_ref, b1_ref, w2_ref, o_ref):
  t = p_ref[0] + p_ref[1]
  dd = dd_ref[0, :] + dd_ref[1, :]
  nd = lax.rsqrt(jnp.maximum(dd, 1.0))
  t = jnp.maximum(t * nd[:, None] + b1_ref[...], 0.0)
  dsum = ds_ref[0, :] + ds_ref[1, :]
  ns = lax.rsqrt(jnp.maximum(dsum, 1.0))
  y = jnp.dot(t, w2_ref[...], preferred_element_type=jnp.float32)
  o_ref[...] = y * ns[:, None]


def _tc2(p, deg_dst_p, deg_src_p, b1, w2):
  return pl.pallas_call(
      _tc2_body,
      grid=(_GRID,),
      in_specs=[
          pl.BlockSpec((NC, _BM, D_H), lambda i: (0, i, 0)),
          pl.BlockSpec((NC, _BM), lambda i: (0, i)),
          pl.BlockSpec((NC, _BM), lambda i: (0, i)),
          pl.BlockSpec((1, D_H), lambda i: (0, 0)),
          pl.BlockSpec((D_H, D_OUT), lambda i: (0, 0)),
      ],
      out_specs=pl.BlockSpec((_BM, D_OUT), lambda i: (i, 0)),
      out_shape=jax.ShapeDtypeStruct((NP, D_OUT), jnp.float32),
  )(p, deg_dst_p, deg_src_p, b1, w2)


def _tc3_body(q_ref, dd_ref, b2_ref, o_ref):
  t = q_ref[0] + q_ref[1]
  dd = dd_ref[0, :] + dd_ref[1, :]
  nd = lax.rsqrt(jnp.maximum(dd, 1.0))
  o_ref[...] = t * nd[:, None] + b2_ref[...]


def _tc3(q, deg_dst_p, b2):
  return pl.pallas_call(
      _tc3_body,
      grid=(_GRID,),
      in_specs=[
          pl.BlockSpec((NC, _BM, D_OUT), lambda i: (0, i, 0)),
          pl.BlockSpec((NC, _BM), lambda i: (0, i)),
          pl.BlockSpec((1, D_OUT), lambda i: (0, 0)),
      ],
      out_specs=pl.BlockSpec((_BM, D_OUT), lambda i: (i, 0)),
      out_shape=jax.ShapeDtypeStruct((NP, D_OUT), jnp.float32),
  )(q, deg_dst_p, b2)


def kernel(in_feat, edge_index, W1, b1, W2, b2):
  pad_ids = jnp.arange(E_PAD - E, dtype=jnp.int32) % (NP - N) + N
  src2d = jnp.concatenate([edge_index[0], pad_ids]).reshape(NW * NB, BB)
  dst2d = jnp.concatenate([edge_index[1], pad_ids]).reshape(NW * NB, BB)
  x_p = jnp.pad(in_feat, ((0, NP - N), (0, 0)))
  degp = _deg_call(src2d, dst2d)
  deg_src_p = degp[:, 0, :]
  deg_dst_p = degp[:, 1, :]
  h1 = _tc1(x_p, W1, deg_src_p)
  p1 = _agg_h(h1, src2d, dst2d)
  h2 = _tc2(p1, deg_dst_p, deg_src_p, b1.reshape(1, -1), W2)
  p2 = _agg_o(h2, src2d, dst2d)
  return _tc3(p2, deg_dst_p, b2.reshape(1, -1))[:N]


# speedup vs baseline: 1.0071x; 1.0020x over previous
import functools

import jax
import jax.numpy as jnp
from jax import lax
from jax.experimental import pallas as pl
from jax.experimental.pallas import tpu as pltpu
from jax.experimental.pallas import tpu_sc as plsc

N = 10000
E = 320000
D_IN = 128
D_H = 128
D_OUT = 64

NC = 2
NS = 16
NW = NC * NS
BB = 128
NB = 80
E_PAD = NW * NB * BB
HB = 40
NP = 10240
RPS = NP // NS

_mesh = plsc.VectorSubcoreMesh(
    core_axis_name="c", subcore_axis_name="s", num_cores=NC, num_subcores=NS
)


def _zero_vmem_2d(ref, rows, cols):
  zv = jnp.zeros((16,), jnp.float32)

  def body(i, carry):
    r = i // (cols // 16)
    c = i % (cols // 16)
    ref[r, pl.ds(c * 16, 16)] = zv
    return carry

  lax.fori_loop(0, rows * (cols // 16), body, 0)


def _deg_body(src_hbm, dst_hbm, out_hbm, sidx_v, didx_v, ones_v, zb_v,
              dsrc_sh, ddst_sh):
  c = lax.axis_index("c")
  s = lax.axis_index("s")
  w = c * NS + s

  def init_body(i, carry):
    zb_v[pl.ds(i * 16, 16)] = jnp.zeros((16,), jnp.float32)
    return carry

  lax.fori_loop(0, RPS // 16, init_body, 0)

  def ones_body(i, carry):
    ones_v[pl.ds(i * 16, 16)] = jnp.ones((16,), jnp.float32)
    return carry

  lax.fori_loop(0, BB // 16, ones_body, 0)

  off = pl.multiple_of(s * RPS, 8)
  pltpu.sync_copy(zb_v, dsrc_sh.at[pl.ds(off, RPS)])
  pltpu.sync_copy(zb_v, ddst_sh.at[pl.ds(off, RPS)])
  pltpu.sync_copy(src_hbm.at[pl.ds(w * NB, NB)], sidx_v)
  pltpu.sync_copy(dst_hbm.at[pl.ds(w * NB, NB)], didx_v)
  plsc.subcore_barrier()

  def body(i, carry):
    pltpu.sync_copy(ones_v, dsrc_sh.at[sidx_v.at[i]], add=True)
    pltpu.sync_copy(ones_v, ddst_sh.at[didx_v.at[i]], add=True)
    return carry

  lax.fori_loop(0, NB, body, 0)
  plsc.subcore_barrier()
  pltpu.sync_copy(dsrc_sh.at[pl.ds(off, RPS)], out_hbm.at[c, 0, pl.ds(off, RPS)])
  pltpu.sync_copy(ddst_sh.at[pl.ds(off, RPS)], out_hbm.at[c, 1, pl.ds(off, RPS)])


_deg_call = pl.kernel(
    _deg_body,
    out_type=jax.ShapeDtypeStruct((NC, 2, NP), jnp.float32),
    mesh=_mesh,
    scratch_types=[
        pltpu.VMEM((NB, BB), jnp.int32),
        pltpu.VMEM((NB, BB), jnp.int32),
        pltpu.VMEM((BB,), jnp.float32),
        pltpu.VMEM((RPS,), jnp.float32),
        pltpu.VMEM_SHARED((NP,), jnp.float32),
        pltpu.VMEM_SHARED((NP,), jnp.float32),
    ],
)


def _agg_body(d, h_hbm, src_hbm, dst_hbm, out_hbm, sidx_v, didx_v, rows_v,
              acc_sh, gsem0, gsem1):
  c = lax.axis_index("c")
  s = lax.axis_index("s")
  w = c * NS + s
  gsems = (gsem0, gsem1)

  _zero_vmem_2d(rows_v.at[0], BB, d)
  for j in range(RPS // BB):
    pltpu.sync_copy(rows_v.at[0], acc_sh.at[pl.ds(s * RPS + j * BB, BB)])
  plsc.subcore_barrier()

  for p in range(NB // HB):
    pltpu.sync_copy(src_hbm.at[pl.ds(w * NB + p * HB, HB)], sidx_v)
    pltpu.sync_copy(dst_hbm.at[pl.ds(w * NB + p * HB, HB)], didx_v)
    pltpu.async_copy(h_hbm.at[sidx_v.at[0]], rows_v.at[0], gsems[0])
    pltpu.async_copy(h_hbm.at[sidx_v.at[1]], rows_v.at[1], gsems[1])

    def body(jj, carry):
      j = jj * 2
      for b in range(2):
        i = j + b
        pltpu.make_async_copy(h_hbm.at[sidx_v.at[i]], rows_v.at[b],
                              gsems[b]).wait()
        pltpu.sync_copy(rows_v.at[b], acc_sh.at[didx_v.at[i]], add=True)

        @pl.when(i + 2 < HB)
        def _():
          pltpu.async_copy(h_hbm.at[sidx_v.at[i + 2]], rows_v.at[b], gsems[b])

      return carry

    lax.fori_loop(0, HB // 2, body, 0)
  plsc.subcore_barrier()
  pltpu.sync_copy(acc_sh.at[pl.ds(s * RPS, RPS)],
                  out_hbm.at[c, pl.ds(s * RPS, RPS)])


def _make_agg(d):
  return pl.kernel(
      functools.partial(_agg_body, d),
      out_type=jax.ShapeDtypeStruct((NC, NP, d), jnp.float32),
      mesh=_mesh,
      scratch_types=[
          pltpu.VMEM((HB, BB), jnp.int32),
          pltpu.VMEM((HB, BB), jnp.int32),
          pltpu.VMEM((2, BB, d), jnp.float32),
          pltpu.VMEM_SHARED((NP, d), jnp.float32),
          pltpu.SemaphoreType.DMA,
          pltpu.SemaphoreType.DMA,
      ],
      compiler_params=pltpu.CompilerParams(use_tc_tiling_on_sc=False),
  )


_agg_h = _make_agg(D_H)
_agg_o = _make_agg(D_OUT)


_BM = 1280
_GRID = NP // _BM


def _tc1a_body(x_ref, w_ref, o_ref):
  o_ref[...] = jnp.dot(x_ref[...], w_ref[...],
                       preferred_element_type=jnp.float32)


def _tc1a(x, w1):
  return pl.pallas_call(
      _tc1a_body,
      grid=(_GRID,),
      in_specs=[
          pl.BlockSpec((_BM, D_IN), lambda i: (i, 0)),
          pl.BlockSpec((D_IN, D_H), lambda i: (0, 0)),
      ],
      out_specs=pl.BlockSpec((_BM, D_H), lambda i: (i, 0)),
      out_shape=jax.ShapeDtypeStruct((NP, D_H), jnp.float32),
  )(x, w1)


def _tc1b_body(m_ref, dp_ref, o_ref):
  d = dp_ref[0, :] + dp_ref[1, :]
  ns = lax.rsqrt(jnp.maximum(d, 1.0))
  o_ref[...] = m_ref[...] * ns[:, None]


def _tc1b(m, deg_src_p):
  return pl.pallas_call(
      _tc1b_body,
      grid=(_GRID,),
      in_specs=[
          pl.BlockSpec((_BM, D_H), lambda i: (i, 0)),
          pl.BlockSpec((NC, _BM), lambda i: (0, i)),
      ],
      out_specs=pl.BlockSpec((_BM, D_H), lambda i: (i, 0)),
      out_shape=jax.ShapeDtypeStruct((NP, D_H), jnp.float32),
  )(m, deg_src_p)


def _tc2_body(p_ref, dd_ref, ds_ref, b1_ref, w2_ref, o_ref):
  t = p_ref[0] + p_ref[1]
  dd = dd_ref[0, :] + dd_ref[1, :]
  nd = lax.rsqrt(jnp.maximum(dd, 1.0))
  t = jnp.maximum(t * nd[:, None] + b1_ref[...], 0.0)
  dsum = ds_ref[0, :] + ds_ref[1, :]
  ns = lax.rsqrt(jnp.maximum(dsum, 1.0))
  y = jnp.dot(t, w2_ref[...], preferred_element_type=jnp.float32)
  o_ref[...] = y * ns[:, None]


def _tc2(p, deg_dst_p, deg_src_p, b1, w2):
  return pl.pallas_call(
      _tc2_body,
      grid=(_GRID,),
      in_specs=[
          pl.BlockSpec((NC, _BM, D_H), lambda i: (0, i, 0)),
          pl.BlockSpec((NC, _BM), lambda i: (0, i)),
          pl.BlockSpec((NC, _BM), lambda i: (0, i)),
          pl.BlockSpec((1, D_H), lambda i: (0, 0)),
          pl.BlockSpec((D_H, D_OUT), lambda i: (0, 0)),
      ],
      out_specs=pl.BlockSpec((_BM, D_OUT), lambda i: (i, 0)),
      out_shape=jax.ShapeDtypeStruct((NP, D_OUT), jnp.float32),
  )(p, deg_dst_p, deg_src_p, b1, w2)


def _tc3_body(q_ref, dd_ref, b2_ref, o_ref):
  t = q_ref[0] + q_ref[1]
  dd = dd_ref[0, :] + dd_ref[1, :]
  nd = lax.rsqrt(jnp.maximum(dd, 1.0))
  o_ref[...] = t * nd[:, None] + b2_ref[...]


def _tc3(q, deg_dst_p, b2):
  return pl.pallas_call(
      _tc3_body,
      grid=(_GRID,),
      in_specs=[
          pl.BlockSpec((NC, _BM, D_OUT), lambda i: (0, i, 0)),
          pl.BlockSpec((NC, _BM), lambda i: (0, i)),
          pl.BlockSpec((1, D_OUT), lambda i: (0, 0)),
      ],
      out_specs=pl.BlockSpec((_BM, D_OUT), lambda i: (i, 0)),
      out_shape=jax.ShapeDtypeStruct((NP, D_OUT), jnp.float32),
  )(q, deg_dst_p, b2)


def kernel(in_feat, edge_index, W1, b1, W2, b2):
  pad_ids = jnp.arange(E_PAD - E, dtype=jnp.int32) % (NP - N) + N
  src2d = jnp.concatenate([edge_index[0], pad_ids]).reshape(NW * NB, BB)
  dst2d = jnp.concatenate([edge_index[1], pad_ids]).reshape(NW * NB, BB)
  x_p = jnp.pad(in_feat, ((0, NP - N), (0, 0)))
  m1 = _tc1a(x_p, W1)
  degp = _deg_call(src2d, dst2d)
  deg_src_p = degp[:, 0, :]
  deg_dst_p = degp[:, 1, :]
  h1 = _tc1b(m1, deg_src_p)
  p1 = _agg_h(h1, src2d, dst2d)
  h2 = _tc2(p1, deg_dst_p, deg_src_p, b1.reshape(1, -1), W2)
  p2 = _agg_o(h2, src2d, dst2d)
  return _tc3(p2, deg_dst_p, b2.reshape(1, -1))[:N]


# speedup vs baseline: 1.0098x; 1.0027x over previous
import functools

import jax
import jax.numpy as jnp
import numpy as np
from jax import lax
from jax.experimental import pallas as pl
from jax.experimental.pallas import tpu as pltpu
from jax.experimental.pallas import tpu_sc as plsc

N = 10000
E = 320000
D_IN = 128
D_H = 128
D_OUT = 64

NC = 2
NS = 16
NW = NC * NS
BB = 128
NB = 80
E_PAD = NW * NB * BB
HB = 40
NP = 10240
RPS = NP // NS

_mesh = plsc.VectorSubcoreMesh(
    core_axis_name="c", subcore_axis_name="s", num_cores=NC, num_subcores=NS
)


def _zero_vmem_2d(ref, rows, cols):
  zv = jnp.zeros((16,), jnp.float32)

  def body(i, carry):
    r = i // (cols // 16)
    c = i % (cols // 16)
    ref[r, pl.ds(c * 16, 16)] = zv
    return carry

  lax.fori_loop(0, rows * (cols // 16), body, 0)


def _deg_body(src_hbm, dst_hbm, out_hbm, sidx_v, didx_v, ones_v, zb_v,
              dsrc_sh, ddst_sh):
  c = lax.axis_index("c")
  s = lax.axis_index("s")
  w = c * NS + s

  def init_body(i, carry):
    zb_v[pl.ds(i * 16, 16)] = jnp.zeros((16,), jnp.float32)
    return carry

  lax.fori_loop(0, RPS // 16, init_body, 0)

  def ones_body(i, carry):
    ones_v[pl.ds(i * 16, 16)] = jnp.ones((16,), jnp.float32)
    return carry

  lax.fori_loop(0, BB // 16, ones_body, 0)

  off = pl.multiple_of(s * RPS, 8)
  pltpu.sync_copy(zb_v, dsrc_sh.at[pl.ds(off, RPS)])
  pltpu.sync_copy(zb_v, ddst_sh.at[pl.ds(off, RPS)])
  pltpu.sync_copy(src_hbm.at[pl.ds(w * NB, NB)], sidx_v)
  pltpu.sync_copy(dst_hbm.at[pl.ds(w * NB, NB)], didx_v)
  plsc.subcore_barrier()

  def body(i, carry):
    pltpu.sync_copy(ones_v, dsrc_sh.at[sidx_v.at[i]], add=True)
    pltpu.sync_copy(ones_v, ddst_sh.at[didx_v.at[i]], add=True)
    return carry

  lax.fori_loop(0, NB, body, 0)
  plsc.subcore_barrier()
  pltpu.sync_copy(dsrc_sh.at[pl.ds(off, RPS)], out_hbm.at[c, 0, pl.ds(off, RPS)])
  pltpu.sync_copy(ddst_sh.at[pl.ds(off, RPS)], out_hbm.at[c, 1, pl.ds(off, RPS)])


_deg_call = pl.kernel(
    _deg_body,
    out_type=jax.ShapeDtypeStruct((NC, 2, NP), jnp.float32),
    mesh=_mesh,
    scratch_types=[
        pltpu.VMEM((NB, BB), jnp.int32),
        pltpu.VMEM((NB, BB), jnp.int32),
        pltpu.VMEM((BB,), jnp.float32),
        pltpu.VMEM((RPS,), jnp.float32),
        pltpu.VMEM_SHARED((NP,), jnp.float32),
        pltpu.VMEM_SHARED((NP,), jnp.float32),
    ],
)


def _agg_body(d, h_hbm, src_hbm, dst_hbm, out_hbm, sidx_v, didx_v, rows_v,
              acc_sh, gsem0, gsem1):
  c = lax.axis_index("c")
  s = lax.axis_index("s")
  w = c * NS + s
  gsems = (gsem0, gsem1)

  _zero_vmem_2d(rows_v.at[0], BB, d)
  for j in range(RPS // BB):
    pltpu.sync_copy(rows_v.at[0], acc_sh.at[pl.ds(s * RPS + j * BB, BB)])
  plsc.subcore_barrier()

  for p in range(NB // HB):
    pltpu.sync_copy(src_hbm.at[pl.ds(w * NB + p * HB, HB)], sidx_v)
    pltpu.sync_copy(dst_hbm.at[pl.ds(w * NB + p * HB, HB)], didx_v)
    pltpu.async_copy(h_hbm.at[sidx_v.at[0]], rows_v.at[0], gsems[0])
    pltpu.async_copy(h_hbm.at[sidx_v.at[1]], rows_v.at[1], gsems[1])

    def body(jj, carry):
      j = jj * 2
      for b in range(2):
        i = j + b
        pltpu.make_async_copy(h_hbm.at[sidx_v.at[i]], rows_v.at[b],
                              gsems[b]).wait()
        pltpu.sync_copy(rows_v.at[b], acc_sh.at[didx_v.at[i]], add=True)

        @pl.when(i + 2 < HB)
        def _():
          pltpu.async_copy(h_hbm.at[sidx_v.at[i + 2]], rows_v.at[b], gsems[b])

      return carry

    lax.fori_loop(0, HB // 2, body, 0)
  plsc.subcore_barrier()
  pltpu.sync_copy(acc_sh.at[pl.ds(s * RPS, RPS)],
                  out_hbm.at[c, pl.ds(s * RPS, RPS)])


def _make_agg(d):
  return pl.kernel(
      functools.partial(_agg_body, d),
      out_type=jax.ShapeDtypeStruct((NC, NP, d), jnp.float32),
      mesh=_mesh,
      scratch_types=[
          pltpu.VMEM((HB, BB), jnp.int32),
          pltpu.VMEM((HB, BB), jnp.int32),
          pltpu.VMEM((2, BB, d), jnp.float32),
          pltpu.VMEM_SHARED((NP, d), jnp.float32),
          pltpu.SemaphoreType.DMA,
          pltpu.SemaphoreType.DMA,
      ],
      compiler_params=pltpu.CompilerParams(use_tc_tiling_on_sc=False),
  )


_agg_h = _make_agg(D_H)
_agg_o = _make_agg(D_OUT)


_BM = 1280
_GRID = NP // _BM


def _tc1a_body(x_ref, w_ref, o_ref):
  o_ref[...] = jnp.dot(x_ref[...], w_ref[...],
                       preferred_element_type=jnp.float32)


def _tc1a(x, w1):
  return pl.pallas_call(
      _tc1a_body,
      grid=(_GRID,),
      in_specs=[
          pl.BlockSpec((_BM, D_IN), lambda i: (i, 0)),
          pl.BlockSpec((D_IN, D_H), lambda i: (0, 0)),
      ],
      out_specs=pl.BlockSpec((_BM, D_H), lambda i: (i, 0)),
      out_shape=jax.ShapeDtypeStruct((N, D_H), jnp.float32),
  )(x, w1)


def _tc1b_body(m_ref, dp_ref, o_ref):
  d = dp_ref[0, 0, :] + dp_ref[1, 0, :]
  ns = lax.rsqrt(jnp.maximum(d, 1.0))
  o_ref[...] = m_ref[...] * ns[:, None]


def _tc1b(m, degp):
  return pl.pallas_call(
      _tc1b_body,
      grid=(_GRID,),
      in_specs=[
          pl.BlockSpec((_BM, D_H), lambda i: (i, 0)),
          pl.BlockSpec((NC, 2, _BM), lambda i: (0, 0, i)),
      ],
      out_specs=pl.BlockSpec((_BM, D_H), lambda i: (i, 0)),
      out_shape=jax.ShapeDtypeStruct((N, D_H), jnp.float32),
  )(m, degp)


def _tc2_body(p_ref, dg_ref, b1_ref, w2_ref, o_ref):
  t = p_ref[0] + p_ref[1]
  dd = dg_ref[0, 1, :] + dg_ref[1, 1, :]
  nd = lax.rsqrt(jnp.maximum(dd, 1.0))
  t = jnp.maximum(t * nd[:, None] + b1_ref[...], 0.0)
  dsum = dg_ref[0, 0, :] + dg_ref[1, 0, :]
  ns = lax.rsqrt(jnp.maximum(dsum, 1.0))
  y = jnp.dot(t, w2_ref[...], preferred_element_type=jnp.float32)
  o_ref[...] = y * ns[:, None]


def _tc2(p, degp, b1, w2):
  return pl.pallas_call(
      _tc2_body,
      grid=(_GRID,),
      in_specs=[
          pl.BlockSpec((NC, _BM, D_H), lambda i: (0, i, 0)),
          pl.BlockSpec((NC, 2, _BM), lambda i: (0, 0, i)),
          pl.BlockSpec((1, D_H), lambda i: (0, 0)),
          pl.BlockSpec((D_H, D_OUT), lambda i: (0, 0)),
      ],
      out_specs=pl.BlockSpec((_BM, D_OUT), lambda i: (i, 0)),
      out_shape=jax.ShapeDtypeStruct((N, D_OUT), jnp.float32),
  )(p, degp, b1, w2)


def _tc3_body(q_ref, dg_ref, b2_ref, o_ref):
  t = q_ref[0] + q_ref[1]
  dd = dg_ref[0, 1, :] + dg_ref[1, 1, :]
  nd = lax.rsqrt(jnp.maximum(dd, 1.0))
  o_ref[...] = t * nd[:, None] + b2_ref[...]


def _tc3(q, degp, b2):
  return pl.pallas_call(
      _tc3_body,
      grid=(_GRID,),
      in_specs=[
          pl.BlockSpec((NC, _BM, D_OUT), lambda i: (0, i, 0)),
          pl.BlockSpec((NC, 2, _BM), lambda i: (0, 0, i)),
          pl.BlockSpec((1, D_OUT), lambda i: (0, 0)),
      ],
      out_specs=pl.BlockSpec((_BM, D_OUT), lambda i: (i, 0)),
      out_shape=jax.ShapeDtypeStruct((N, D_OUT), jnp.float32),
  )(q, degp, b2)


_PAD_LO = np.arange(E_PAD - E, dtype=np.int32) % (NP - N)
_PAD_HI = _PAD_LO + N


def kernel(in_feat, edge_index, W1, b1, W2, b2):
  src_g2d = jnp.concatenate([edge_index[0], jnp.asarray(_PAD_LO)]
                            ).reshape(NW * NB, BB)
  src_d2d = jnp.concatenate([edge_index[0], jnp.asarray(_PAD_HI)]
                            ).reshape(NW * NB, BB)
  dst2d = jnp.concatenate([edge_index[1], jnp.asarray(_PAD_HI)]
                          ).reshape(NW * NB, BB)
  m1 = _tc1a(in_feat, W1)
  degp = _deg_call(src_d2d, dst2d)
  h1 = _tc1b(m1, degp)
  p1 = _agg_h(h1, src_g2d, dst2d)
  h2 = _tc2(p1, degp, b1.reshape(1, -1), W2)
  p2 = _agg_o(h2, src_g2d, dst2d)
  return _tc3(p2, degp, b2.reshape(1, -1))


# speedup vs baseline: 1.0425x; 1.0325x over previous
import functools

import jax
import jax.numpy as jnp
import numpy as np
from jax import lax
from jax.experimental import pallas as pl
from jax.experimental.pallas import tpu as pltpu
from jax.experimental.pallas import tpu_sc as plsc

N = 10000
E = 320000
D_IN = 128
D_H = 128
D_OUT = 64

NC = 2
NS = 16
NW = NC * NS
BB = 128
NB = 80
E_PAD = NW * NB * BB
HB = 40
NP = 10240
RPS = NP // NS
ER = E // BB
RW = ER - (NW - 1) * NB
PR = NB - RW

_mesh = plsc.VectorSubcoreMesh(
    core_axis_name="c", subcore_axis_name="s", num_cores=NC, num_subcores=NS
)


def _zero_vmem_2d(ref, rows, cols):
  zv = jnp.zeros((16,), jnp.float32)

  def body(i, carry):
    r = i // (cols // 16)
    c = i % (cols // 16)
    ref[r, pl.ds(c * 16, 16)] = zv
    return carry

  lax.fori_loop(0, rows * (cols // 16), body, 0)


def _deg_body(edges_hbm, pads_hbm, out_hbm, sidx_v, didx_v, ones_v, zb_v,
              dsrc_sh, ddst_sh):
  c = lax.axis_index("c")
  s = lax.axis_index("s")
  w = c * NS + s

  def init_body(i, carry):
    zb_v[pl.ds(i * 16, 16)] = jnp.zeros((16,), jnp.float32)
    return carry

  lax.fori_loop(0, RPS // 16, init_body, 0)

  def ones_body(i, carry):
    ones_v[pl.ds(i * 16, 16)] = jnp.ones((16,), jnp.float32)
    return carry

  lax.fori_loop(0, BB // 16, ones_body, 0)

  off = pl.multiple_of(s * RPS, 8)
  pltpu.sync_copy(zb_v, dsrc_sh.at[pl.ds(off, RPS)])
  pltpu.sync_copy(zb_v, ddst_sh.at[pl.ds(off, RPS)])

  @pl.when(w < NW - 1)
  def _():
    pltpu.sync_copy(edges_hbm.at[0, pl.ds(w * NB, NB)], sidx_v)
    pltpu.sync_copy(edges_hbm.at[1, pl.ds(w * NB, NB)], didx_v)

  @pl.when(w == NW - 1)
  def _():
    pltpu.sync_copy(edges_hbm.at[0, pl.ds((NW - 1) * NB, RW)],
                    sidx_v.at[pl.ds(0, RW)])
    pltpu.sync_copy(pads_hbm.at[0], sidx_v.at[pl.ds(RW, PR)])
    pltpu.sync_copy(edges_hbm.at[1, pl.ds((NW - 1) * NB, RW)],
                    didx_v.at[pl.ds(0, RW)])
    pltpu.sync_copy(pads_hbm.at[1], didx_v.at[pl.ds(RW, PR)])

  plsc.subcore_barrier()

  def body(i, carry):
    pltpu.sync_copy(ones_v, dsrc_sh.at[sidx_v.at[i]], add=True)
    pltpu.sync_copy(ones_v, ddst_sh.at[didx_v.at[i]], add=True)
    return carry

  lax.fori_loop(0, NB, body, 0)
  plsc.subcore_barrier()
  pltpu.sync_copy(dsrc_sh.at[pl.ds(off, RPS)], out_hbm.at[c, 0, pl.ds(off, RPS)])
  pltpu.sync_copy(ddst_sh.at[pl.ds(off, RPS)], out_hbm.at[c, 1, pl.ds(off, RPS)])


_deg_call = pl.kernel(
    _deg_body,
    out_type=jax.ShapeDtypeStruct((NC, 2, NP), jnp.float32),
    compiler_params=pltpu.CompilerParams(use_tc_tiling_on_sc=False),
    mesh=_mesh,
    scratch_types=[
        pltpu.VMEM((NB, BB), jnp.int32),
        pltpu.VMEM((NB, BB), jnp.int32),
        pltpu.VMEM((BB,), jnp.float32),
        pltpu.VMEM((RPS,), jnp.float32),
        pltpu.VMEM_SHARED((NP,), jnp.float32),
        pltpu.VMEM_SHARED((NP,), jnp.float32),
    ],
)


def _agg_body(d, h_hbm, edges_hbm, pads_hbm, out_hbm, sidx_v, didx_v, rows_v,
              acc_sh, gsem0, gsem1):
  c = lax.axis_index("c")
  s = lax.axis_index("s")
  w = c * NS + s
  gsems = (gsem0, gsem1)

  _zero_vmem_2d(rows_v.at[0], BB, d)
  for j in range(RPS // BB):
    pltpu.sync_copy(rows_v.at[0], acc_sh.at[pl.ds(s * RPS + j * BB, BB)])
  plsc.subcore_barrier()

  for p in range(NB // HB):

    @pl.when(w < NW - 1)
    def _():
      pltpu.sync_copy(edges_hbm.at[0, pl.ds(w * NB + p * HB, HB)], sidx_v)
      pltpu.sync_copy(edges_hbm.at[1, pl.ds(w * NB + p * HB, HB)], didx_v)

    @pl.when(w == NW - 1)
    def _():
      if p == 0:
        pltpu.sync_copy(edges_hbm.at[0, pl.ds((NW - 1) * NB, RW)],
                        sidx_v.at[pl.ds(0, RW)])
        pltpu.sync_copy(pads_hbm.at[0, pl.ds(0, HB - RW)],
                        sidx_v.at[pl.ds(RW, HB - RW)])
        pltpu.sync_copy(edges_hbm.at[1, pl.ds((NW - 1) * NB, RW)],
                        didx_v.at[pl.ds(0, RW)])
        pltpu.sync_copy(pads_hbm.at[1, pl.ds(0, HB - RW)],
                        didx_v.at[pl.ds(RW, HB - RW)])
      else:
        pltpu.sync_copy(pads_hbm.at[0, pl.ds(p * HB - RW, HB)], sidx_v)
        pltpu.sync_copy(pads_hbm.at[1, pl.ds(p * HB - RW, HB)], didx_v)
    pltpu.async_copy(h_hbm.at[sidx_v.at[0]], rows_v.at[0], gsems[0])
    pltpu.async_copy(h_hbm.at[sidx_v.at[1]], rows_v.at[1], gsems[1])

    def body(jj, carry):
      j = jj * 2
      for b in range(2):
        i = j + b
        pltpu.make_async_copy(h_hbm.at[sidx_v.at[i]], rows_v.at[b],
                              gsems[b]).wait()
        pltpu.sync_copy(rows_v.at[b], acc_sh.at[didx_v.at[i]], add=True)

        @pl.when(i + 2 < HB)
        def _():
          pltpu.async_copy(h_hbm.at[sidx_v.at[i + 2]], rows_v.at[b], gsems[b])

      return carry

    lax.fori_loop(0, HB // 2, body, 0)
  plsc.subcore_barrier()
  pltpu.sync_copy(acc_sh.at[pl.ds(s * RPS, RPS)],
                  out_hbm.at[c, pl.ds(s * RPS, RPS)])


def _make_agg(d):
  return pl.kernel(
      functools.partial(_agg_body, d),
      out_type=jax.ShapeDtypeStruct((NC, NP, d), jnp.float32),
      mesh=_mesh,
      scratch_types=[
          pltpu.VMEM((HB, BB), jnp.int32),
          pltpu.VMEM((HB, BB), jnp.int32),
          pltpu.VMEM((2, BB, d), jnp.float32),
          pltpu.VMEM_SHARED((NP, d), jnp.float32),
          pltpu.SemaphoreType.DMA,
          pltpu.SemaphoreType.DMA,
      ],
      compiler_params=pltpu.CompilerParams(use_tc_tiling_on_sc=False),
  )


_agg_h = _make_agg(D_H)
_agg_o = _make_agg(D_OUT)


_BM = 1280
_GRID = NP // _BM


def _tc1a_body(x_ref, w_ref, o_ref):
  o_ref[...] = jnp.dot(x_ref[...], w_ref[...],
                       preferred_element_type=jnp.float32)


def _tc1a(x, w1):
  return pl.pallas_call(
      _tc1a_body,
      grid=(_GRID,),
      in_specs=[
          pl.BlockSpec((_BM, D_IN), lambda i: (i, 0)),
          pl.BlockSpec((D_IN, D_H), lambda i: (0, 0)),
      ],
      out_specs=pl.BlockSpec((_BM, D_H), lambda i: (i, 0)),
      out_shape=jax.ShapeDtypeStruct((N, D_H), jnp.float32),
  )(x, w1)


def _tc1b_body(m_ref, dp_ref, o_ref):
  d = dp_ref[0, 0, :] + dp_ref[1, 0, :]
  ns = lax.rsqrt(jnp.maximum(d, 1.0))
  o_ref[...] = m_ref[...] * ns[:, None]


def _tc1b(m, degp):
  return pl.pallas_call(
      _tc1b_body,
      grid=(_GRID,),
      in_specs=[
          pl.BlockSpec((_BM, D_H), lambda i: (i, 0)),
          pl.BlockSpec((NC, 2, _BM), lambda i: (0, 0, i)),
      ],
      out_specs=pl.BlockSpec((_BM, D_H), lambda i: (i, 0)),
      out_shape=jax.ShapeDtypeStruct((N, D_H), jnp.float32),
  )(m, degp)


def _tc2_body(p_ref, dg_ref, b1_ref, w2_ref, o_ref):
  t = p_ref[0] + p_ref[1]
  dd = dg_ref[0, 1, :] + dg_ref[1, 1, :]
  nd = lax.rsqrt(jnp.maximum(dd, 1.0))
  t = jnp.maximum(t * nd[:, None] + b1_ref[...], 0.0)
  dsum = dg_ref[0, 0, :] + dg_ref[1, 0, :]
  ns = lax.rsqrt(jnp.maximum(dsum, 1.0))
  y = jnp.dot(t, w2_ref[...], preferred_element_type=jnp.float32)
  o_ref[...] = y * ns[:, None]


def _tc2(p, degp, b1, w2):
  return pl.pallas_call(
      _tc2_body,
      grid=(_GRID,),
      in_specs=[
          pl.BlockSpec((NC, _BM, D_H), lambda i: (0, i, 0)),
          pl.BlockSpec((NC, 2, _BM), lambda i: (0, 0, i)),
          pl.BlockSpec((1, D_H), lambda i: (0, 0)),
          pl.BlockSpec((D_H, D_OUT), lambda i: (0, 0)),
      ],
      out_specs=pl.BlockSpec((_BM, D_OUT), lambda i: (i, 0)),
      out_shape=jax.ShapeDtypeStruct((N, D_OUT), jnp.float32),
  )(p, degp, b1, w2)


def _tc3_body(q_ref, dg_ref, b2_ref, o_ref):
  t = q_ref[0] + q_ref[1]
  dd = dg_ref[0, 1, :] + dg_ref[1, 1, :]
  nd = lax.rsqrt(jnp.maximum(dd, 1.0))
  o_ref[...] = t * nd[:, None] + b2_ref[...]


def _tc3(q, degp, b2):
  return pl.pallas_call(
      _tc3_body,
      grid=(_GRID,),
      in_specs=[
          pl.BlockSpec((NC, _BM, D_OUT), lambda i: (0, i, 0)),
          pl.BlockSpec((NC, 2, _BM), lambda i: (0, 0, i)),
          pl.BlockSpec((1, D_OUT), lambda i: (0, 0)),
      ],
      out_specs=pl.BlockSpec((_BM, D_OUT), lambda i: (i, 0)),
      out_shape=jax.ShapeDtypeStruct((N, D_OUT), jnp.float32),
  )(q, degp, b2)


_PAD_LO = (np.arange(PR * BB, dtype=np.int32) % (NP - N)).reshape(PR, BB)
_PAD_HI = _PAD_LO + N
_PADS_DEG = np.stack([_PAD_HI, _PAD_HI])
_PADS_AGG = np.stack([_PAD_LO, _PAD_HI])


def kernel(in_feat, edge_index, W1, b1, W2, b2):
  edges3 = edge_index.reshape(2, ER, BB)
  pads_deg = jnp.asarray(_PADS_DEG)
  pads_agg = jnp.asarray(_PADS_AGG)
  m1 = _tc1a(in_feat, W1)
  degp = _deg_call(edges3, pads_deg)
  h1 = _tc1b(m1, degp)
  p1 = _agg_h(h1, edges3, pads_agg)
  h2 = _tc2(p1, degp, b1.reshape(1, -1), W2)
  p2 = _agg_o(h2, edges3, pads_agg)
  return _tc3(p2, degp, b2.reshape(1, -1))
